# baseline (speedup 1.0000x reference)
import functools

import jax
import jax.numpy as jnp
from jax.experimental import pallas as pl
from jax.experimental.pallas import tpu as pltpu

N = 512
B = 16
H = 32
C = 2
SEQ = 12
HALF = SEQ // 2
CIN = C + H
NTERM = 5
GB = 4
NG = B // GB


def _mm(a, b):
    return jax.lax.dot_general(a, b, (((1,), (0,)), ((), ())),
                               preferred_element_type=jnp.float32)


def _bf(v):
    return v.astype(jnp.bfloat16)


def _dcrnn_half(xT_ref, s0_ref, a01_ref, mgc_ref,
                w4g_ref, w4c_ref, bgct_ref, out_ref, sfin_ref,
                x1_ref, x2_ref, x3_ref, x4_ref):
    A01 = a01_ref[...]
    A0 = A01[:N]
    A1 = A01[N:]

    x0 = xT_ref[...]
    x13 = _mm(A01, x0)
    x1_ref[...] = _bf(x13[:N])
    x3_ref[...] = _bf(x13[N:])
    x2_ref[...] = _bf(_mm(A0, x1_ref[...]))
    x4_ref[...] = _bf(_mm(A1, x3_ref[...]))

    Mgc = mgc_ref[...]
    W4g = w4g_ref[...]
    W4c = w4c_ref[...]
    bgct = bgct_ref[...]

    S = s0_ref[...]
    for t in range(HALF):
        sl = slice(t * B * C, (t + 1) * B * C)
        Xcat = jnp.concatenate(
            [xT_ref[:, sl], x1_ref[:, sl], x2_ref[:, sl],
             x3_ref[:, sl], x4_ref[:, sl]], axis=1)
        GxCx = _mm(Xcat, Mgc) + bgct
        Gx = GxCx[:, :B * 2 * H]
        Cx = GxCx[:, B * 2 * H:]

        Sb = _bf(S)
        H13 = _mm(A01, Sb)
        H1 = _bf(H13[:N])
        H3 = _bf(H13[N:])
        H2 = _bf(_mm(A0, H1))
        H4 = _bf(_mm(A1, H3))
        z_parts = []
        r_parts = []
        for g in range(NG):
            gl = slice(g * GB * H, (g + 1) * GB * H)
            Hcat = jnp.concatenate(
                [Sb[:, gl], H1[:, gl], H2[:, gl], H3[:, gl], H4[:, gl]],
                axis=1)
            act = jax.nn.sigmoid(_mm(Hcat, W4g)
                                 + Gx[:, g * 256:(g + 1) * 256])
            z_parts.append(act[:, :128])
            r_parts.append(act[:, 128:])
        Z = jnp.concatenate(z_parts, axis=1)
        R = jnp.concatenate(r_parts, axis=1)

        Cst = Z * S
        Cb = _bf(Cst)
        G13 = _mm(A01, Cb)
        G1 = _bf(G13[:N])
        G3 = _bf(G13[N:])
        G2 = _bf(_mm(A0, G1))
        G4 = _bf(_mm(A1, G3))
        hc_parts = []
        for g in range(NG):
            gl = slice(g * GB * H, (g + 1) * GB * H)
            Ccat = jnp.concatenate(
                [Cb[:, gl], G1[:, gl], G2[:, gl], G3[:, gl], G4[:, gl]],
                axis=1)
            hc_parts.append(jnp.tanh(_mm(Ccat, W4c) + Cx[:, gl]))
        HC = jnp.concatenate(hc_parts, axis=1)

        S = R * S + (1.0 - R) * HC
        out_ref[t] = S
    sfin_ref[...] = S


@jax.jit
def kernel(x, init_state, A0, A1, Wg, bg, Wc, bc):
    f32 = jnp.float32
    bf16 = jnp.bfloat16
    xT = x.transpose(2, 1, 0, 3).reshape(N, SEQ * B * C)
    S0 = init_state[0].transpose(1, 0, 2).reshape(N, B * H)

    Wgx = jnp.stack([Wg[CIN * k:CIN * k + C] for k in range(NTERM)])
    Wcx = jnp.stack([Wc[CIN * k:CIN * k + C] for k in range(NTERM)])
    Wgh = jnp.stack([Wg[CIN * k + C:CIN * (k + 1)]
                     for k in range(NTERM)])
    Wch = jnp.stack([Wc[CIN * k + C:CIN * (k + 1)]
                     for k in range(NTERM)])

    eye4 = jnp.eye(GB, dtype=f32)
    Wz = Wgh[:, :, :H]
    Wr = Wgh[:, :, H:]
    Tz = jnp.einsum('khj,ab->kahbj', Wz, eye4).reshape(NTERM * GB * H,
                                                       GB * H)
    Tr = jnp.einsum('khj,ab->kahbj', Wr, eye4).reshape(NTERM * GB * H,
                                                       GB * H)
    W4g = jnp.concatenate([Tz, Tr], axis=1)
    W4c = jnp.einsum('khj,ab->kahbj', Wch, eye4).reshape(NTERM * GB * H,
                                                         GB * H)

    eyeB = jnp.eye(B, dtype=f32)
    Mg0 = jnp.einsum('kcd,ab->kacbd', Wgx, eyeB).reshape(NTERM * B * C,
                                                         B * 2 * H)
    perm = [(b // GB) * 2 * GB * H + (d // H) * GB * H + (b % GB) * H + d % H
            for b in range(B) for d in range(2 * H)]
    inv = [0] * (B * 2 * H)
    for old, new in enumerate(perm):
        inv[new] = old
    Mg = Mg0[:, jnp.array(inv)]
    Mc = jnp.einsum('kcd,ab->kacbd', Wcx, eyeB).reshape(NTERM * B * C,
                                                        B * H)
    bz = jnp.tile(bg[:H], GB)
    br = jnp.tile(bg[H:], GB)
    bgt = jnp.tile(jnp.concatenate([bz, br]), NG).reshape(1, B * 2 * H)
    bct = jnp.tile(bc, B).reshape(1, B * H)

    A01 = jnp.concatenate([A0, A1], axis=0)
    Mgc = jnp.concatenate([Mg, Mc], axis=1)
    bgct = jnp.concatenate([bgt, bct], axis=1)

    call = pl.pallas_call(
        _dcrnn_half,
        out_shape=[jax.ShapeDtypeStruct((HALF, N, B * H), f32),
                   jax.ShapeDtypeStruct((N, B * H), f32)],
        scratch_shapes=[pltpu.VMEM((N, HALF * B * C), bf16)] * 4,
    )
    consts = (A01.astype(bf16), Mgc.astype(bf16),
              W4g.astype(bf16), W4c.astype(bf16), bgct)
    xTb = xT.astype(bf16)
    Y0, S6 = call(xTb[:, :HALF * B * C], S0, *consts)
    Y1, S12 = call(xTb[:, HALF * B * C:], S6, *consts)

    def _tr(Y):
        return Y.reshape(HALF, N, B, H).transpose(2, 0, 1, 3)

    current = jnp.concatenate([_tr(Y0), _tr(Y1)], axis=1)
    hiddens = S12.reshape(N, B, H).transpose(1, 0, 2)[None]
    return (current, hiddens)


# speedup vs baseline: 1.1814x; 1.1814x over previous
import jax
import jax.numpy as jnp
from jax.experimental import pallas as pl
from jax.experimental.pallas import tpu as pltpu

N = 512
B = 16
H = 32
C = 2
SEQ = 12
CIN = C + H
NTERM = 5
GB = 4
NG = B // GB


def _mm(a, b):
    return jax.lax.dot_general(a, b, (((1,), (0,)), ((), ())),
                               preferred_element_type=jnp.float32)


def _bf(v):
    return v.astype(jnp.bfloat16)


def _dcrnn_kernel(xT_ref, s0_ref, a01_ref, mgc_ref,
                  w4g_ref, w4c_ref, bgct_ref, out_ref,
                  x1_ref, x2_ref, x3_ref, x4_ref):
    A01 = a01_ref[...]
    A0 = A01[:N]
    A1 = A01[N:]

    x0 = xT_ref[...]
    x13 = _mm(A01, x0)
    x1_ref[...] = _bf(x13[:N])
    x3_ref[...] = _bf(x13[N:])
    x2_ref[...] = _bf(_mm(A0, x1_ref[...]))
    x4_ref[...] = _bf(_mm(A1, x3_ref[...]))

    Mgc = mgc_ref[...]
    W4g = w4g_ref[...]
    W4c = w4c_ref[...]
    bgct = bgct_ref[...]

    S = s0_ref[...]
    for t in range(SEQ):
        sl = slice(t * B * C, (t + 1) * B * C)
        Xcat = jnp.concatenate(
            [xT_ref[:, sl], x1_ref[:, sl], x2_ref[:, sl],
             x3_ref[:, sl], x4_ref[:, sl]], axis=1)
        GxCx = _mm(Xcat, Mgc) + bgct
        Gx = GxCx[:, :B * 2 * H]
        Cx = GxCx[:, B * 2 * H:]

        Sb = _bf(S)
        H13 = _mm(A01, Sb)
        H1 = _bf(H13[:N])
        H3 = _bf(H13[N:])
        H2 = _bf(_mm(A0, H1))
        H4 = _bf(_mm(A1, H3))
        z_parts = []
        r_parts = []
        for g in range(NG):
            gl = slice(g * GB * H, (g + 1) * GB * H)
            Hcat = jnp.concatenate(
                [Sb[:, gl], H1[:, gl], H2[:, gl], H3[:, gl], H4[:, gl]],
                axis=1)
            act = jax.nn.sigmoid(_mm(Hcat, W4g)
                                 + Gx[:, g * 256:(g + 1) * 256])
            z_parts.append(act[:, :128])
            r_parts.append(act[:, 128:])
        Z = jnp.concatenate(z_parts, axis=1)
        R = jnp.concatenate(r_parts, axis=1)

        Cst = Z * S
        Cb = _bf(Cst)
        G13 = _mm(A01, Cb)
        G1 = _bf(G13[:N])
        G3 = _bf(G13[N:])
        G2 = _bf(_mm(A0, G1))
        G4 = _bf(_mm(A1, G3))
        hc_parts = []
        for g in range(NG):
            gl = slice(g * GB * H, (g + 1) * GB * H)
            Ccat = jnp.concatenate(
                [Cb[:, gl], G1[:, gl], G2[:, gl], G3[:, gl], G4[:, gl]],
                axis=1)
            hc_parts.append(jnp.tanh(_mm(Ccat, W4c) + Cx[:, gl]))
        HC = jnp.concatenate(hc_parts, axis=1)

        S = R * S + (1.0 - R) * HC
        out_ref[t] = S


@jax.jit
def kernel(x, init_state, A0, A1, Wg, bg, Wc, bc):
    f32 = jnp.float32
    bf16 = jnp.bfloat16
    xT = x.transpose(2, 1, 0, 3).reshape(N, SEQ * B * C)
    S0 = init_state[0].transpose(1, 0, 2).reshape(N, B * H)

    Wgx = jnp.stack([Wg[CIN * k:CIN * k + C] for k in range(NTERM)])
    Wcx = jnp.stack([Wc[CIN * k:CIN * k + C] for k in range(NTERM)])
    Wgh = jnp.stack([Wg[CIN * k + C:CIN * (k + 1)]
                     for k in range(NTERM)])
    Wch = jnp.stack([Wc[CIN * k + C:CIN * (k + 1)]
                     for k in range(NTERM)])

    eye4 = jnp.eye(GB, dtype=f32)
    Wz = Wgh[:, :, :H]
    Wr = Wgh[:, :, H:]
    Tz = jnp.einsum('khj,ab->kahbj', Wz, eye4).reshape(NTERM * GB * H,
                                                       GB * H)
    Tr = jnp.einsum('khj,ab->kahbj', Wr, eye4).reshape(NTERM * GB * H,
                                                       GB * H)
    W4g = jnp.concatenate([Tz, Tr], axis=1)
    W4c = jnp.einsum('khj,ab->kahbj', Wch, eye4).reshape(NTERM * GB * H,
                                                         GB * H)

    eyeB = jnp.eye(B, dtype=f32)
    Mg0 = jnp.einsum('kcd,ab->kacbd', Wgx, eyeB).reshape(NTERM * B * C,
                                                         B * 2 * H)
    perm = [(b // GB) * 2 * GB * H + (d // H) * GB * H + (b % GB) * H + d % H
            for b in range(B) for d in range(2 * H)]
    inv = [0] * (B * 2 * H)
    for old, new in enumerate(perm):
        inv[new] = old
    Mg = Mg0[:, jnp.array(inv)]
    Mc = jnp.einsum('kcd,ab->kacbd', Wcx, eyeB).reshape(NTERM * B * C,
                                                        B * H)
    bz = jnp.tile(bg[:H], GB)
    br = jnp.tile(bg[H:], GB)
    bgt = jnp.tile(jnp.concatenate([bz, br]), NG).reshape(1, B * 2 * H)
    bct = jnp.tile(bc, B).reshape(1, B * H)

    A01 = jnp.concatenate([A0, A1], axis=0)
    Mgc = jnp.concatenate([Mg, Mc], axis=1)
    bgct = jnp.concatenate([bgt, bct], axis=1)

    Y = pl.pallas_call(
        _dcrnn_kernel,
        out_shape=jax.ShapeDtypeStruct((SEQ, N, B * H), f32),
        scratch_shapes=[pltpu.VMEM((N, SEQ * B * C), bf16)] * 4,
    )(xT.astype(bf16), S0, A01.astype(bf16), Mgc.astype(bf16),
      W4g.astype(bf16), W4c.astype(bf16), bgct)

    current = Y.reshape(SEQ, N, B, H).transpose(2, 0, 1, 3)
    hiddens = current[:, -1][None]
    return (current, hiddens)
